# baseline (speedup 1.0000x reference)
import functools

import jax
import jax.numpy as jnp
from jax import lax
from jax.experimental import pallas as pl
from jax.experimental.pallas import tpu as pltpu
from jax.experimental.pallas import tpu_sc as plsc


_CHUNK = 128


@functools.lru_cache(maxsize=None)
def _make_sc_gather(V, D, B_TOT):
    info = plsc.get_sparse_core_info()
    NC, NS = info.num_cores, info.num_subcores
    NW = NC * NS
    b_per_w = B_TOT // NW
    n_chunks = b_per_w // _CHUNK
    assert b_per_w % _CHUNK == 0 and B_TOT % NW == 0

    mesh = plsc.VectorSubcoreMesh(core_axis_name="c", subcore_axis_name="s")

    @functools.partial(
        pl.kernel,
        out_type=jax.ShapeDtypeStruct((B_TOT, D), jnp.float32),
        mesh=mesh,
        scratch_types=[
            pltpu.VMEM((n_chunks, _CHUNK), jnp.int32),
            pltpu.VMEM((b_per_w, D), jnp.float32),
            pltpu.SemaphoreType.DMA,
        ],
        compiler_params=pltpu.CompilerParams(use_tc_tiling_on_sc=False),
    )
    def sc_gather(table_hbm, idx_hbm, out_hbm, idx_v, rows_v, sem):
        wid = lax.axis_index("s") * NC + lax.axis_index("c")
        pltpu.sync_copy(idx_hbm.at[wid], idx_v)
        copies = []
        for j in range(n_chunks):
            copies.append(
                pltpu.async_copy(
                    table_hbm.at[idx_v.at[j]],
                    rows_v.at[pl.ds(j * _CHUNK, _CHUNK)],
                    sem,
                )
            )
        for cp in copies:
            cp.wait()
        pltpu.sync_copy(rows_v, out_hbm.at[pl.ds(wid * b_per_w, b_per_w)])

    return sc_gather


_VT = 512


def _head_body(x_ref, p_ref, w_ref, b_ref, o_ref):
    x = x_ref[...] + p_ref[...]
    o_ref[...] = (
        jnp.dot(x, w_ref[...], preferred_element_type=jnp.float32) + b_ref[...]
    )


@functools.lru_cache(maxsize=None)
def _make_tc_head(B_TOT, D, V):
    nv = pl.cdiv(V, _VT)
    return pl.pallas_call(
        _head_body,
        grid=(nv,),
        in_specs=[
            pl.BlockSpec((B_TOT, D), lambda v: (0, 0)),
            pl.BlockSpec((B_TOT, D), lambda v: (0, 0)),
            pl.BlockSpec((D, _VT), lambda v: (0, v)),
            pl.BlockSpec((1, _VT), lambda v: (0, v)),
        ],
        out_specs=pl.BlockSpec((B_TOT, _VT), lambda v: (0, v)),
        out_shape=jax.ShapeDtypeStruct((B_TOT, V), jnp.float32),
    )


def kernel(idx, token_table, pos_table, W, b):
    B, T = idx.shape
    V, D = token_table.shape
    B_TOT = B * T

    idx_flat = idx.reshape(-1).astype(jnp.int32)
    info = plsc.get_sparse_core_info()
    NW = info.num_cores * info.num_subcores
    idx3 = idx_flat.reshape(NW, (B_TOT // NW) // _CHUNK, _CHUNK)

    x = _make_sc_gather(V, D, B_TOT)(token_table, idx3)

    pos_tiled = jnp.tile(pos_table, (B_TOT // T, 1))
    logits2d = _make_tc_head(B_TOT, D, V)(
        x, pos_tiled, W, b.reshape(1, V)
    )
    return logits2d.reshape(B, T, V)
